# baseline (speedup 1.0000x reference)
import functools

import jax
import jax.numpy as jnp
from jax import lax
from jax.experimental import pallas as pl
from jax.experimental.pallas import tpu as pltpu
from jax.experimental.pallas import tpu_sc as plsc

B, N, M = 16, 2048, 64
CUTOFF = 6.0
NUM_WORKERS = 32
ATOMS_PER_W = (B * N) // NUM_WORKERS
CHUNK = 128
NCHUNK = ATOMS_PER_W // CHUNK
GROUPS = CHUNK // 16
UNROLL = 8


def _sc_stage(d_hbm, pt_hbm, coef_hbm, phi_hbm, rho_hbm,
              da, db, pa, pb, coef_v, phi_v, rho_v, sem_a, sem_b):
    wid = lax.axis_index("c") * 16 + lax.axis_index("s")
    lane = lax.iota(jnp.int32, 16)
    b_idx = wid // 2
    nbase = (wid % 2) * ATOMS_PER_W

    pltpu.sync_copy(coef_hbm, coef_v)

    zero16 = jnp.zeros((16,), jnp.float32)

    def start(dbuf, pbuf, sem, c):
        n0 = nbase + c * CHUNK
        pltpu.make_async_copy(d_hbm.at[b_idx, pl.ds(n0, CHUNK)], dbuf,
                              sem).start()
        pltpu.make_async_copy(pt_hbm.at[b_idx, pl.ds(n0, CHUNK)], pbuf,
                              sem).start()

    def wait(dbuf, pbuf, sem):
        pltpu.make_async_copy(d_hbm.at[0, pl.ds(0, CHUNK)], dbuf, sem).wait()
        pltpu.make_async_copy(pt_hbm.at[0, pl.ds(0, CHUNK)], pbuf, sem).wait()

    def compute(dbuf, pbuf, c):
        for g in range(GROUPS):
            a_vec = g * 16 + lane

            def body(jj, acc):
                ap, ar = acc
                for u in range(UNROLL):
                    j = jnp.zeros((16,), jnp.int32) + (jj * UNROLL + u)
                    d = plsc.load_gather(dbuf, [a_vec, j])
                    pt = plsc.load_gather(pbuf, [a_vec, j])
                    ca1 = plsc.load_gather(coef_v, [pt])
                    cb1 = plsc.load_gather(coef_v, [pt + 3])
                    ca2 = plsc.load_gather(coef_v, [pt + 6])
                    cb2 = plsc.load_gather(coef_v, [pt + 9])
                    off = plsc.load_gather(coef_v, [pt + 12])
                    m = d < CUTOFF
                    phi = ca1 * jnp.exp(cb1 * d) - off
                    rho = ca2 * jnp.exp(cb2 * d)
                    ap = ap + jnp.where(m, phi, 0.0)
                    ar = ar + jnp.where(m, rho, 0.0)
                return ap, ar

            ap, ar = lax.fori_loop(0, M // UNROLL, body, (zero16, zero16))
            oidx = c * CHUNK + g * 16 + lane
            plsc.store_scatter(phi_v, [oidx], ap)
            plsc.store_scatter(rho_v, [oidx], ar)

    start(da, pa, sem_a, 0)

    def pair_body(c2, carry):
        c = c2 * 2
        start(db, pb, sem_b, c + 1)
        wait(da, pa, sem_a)
        compute(da, pa, c)
        start(da, pa, sem_a, lax.min(c + 2, NCHUNK - 1))
        wait(db, pb, sem_b)
        compute(db, pb, c + 1)
        return carry

    lax.fori_loop(0, NCHUNK // 2, pair_body, 0)
    wait(da, pa, sem_a)

    pltpu.sync_copy(phi_v, phi_hbm.at[pl.ds(wid * ATOMS_PER_W, ATOMS_PER_W)])
    pltpu.sync_copy(rho_v, rho_hbm.at[pl.ds(wid * ATOMS_PER_W, ATOMS_PER_W)])


def _tc_finale(phi_ref, rho_ref, t_ref, n_ref, f_ref, out_ref):
    phi = phi_ref[...]
    rho = rho_ref[...]
    t = t_ref[...]
    f0 = f_ref[0, 0]
    f1 = f_ref[1, 0]
    fs = jnp.where(t == 0, f0, f1)
    emb = -fs * jnp.sqrt(rho + 1e-12)
    col = lax.broadcasted_iota(jnp.int32, (B, N), 1)
    n = n_ref[:, 0:1]
    amask = col < n
    ae = jnp.where(amask, 0.5 * phi + emb, 0.0)
    e = jnp.sum(ae, axis=1, keepdims=True)
    epa = e / n.astype(jnp.float32)
    out_ref[...] = jnp.broadcast_to(epa, (B, 128))


def kernel(types, pair_types, distances, n_atoms, A, p, q, xi, r0, F_scale,
           offsets):
    f32 = jnp.float32
    ca1 = (A * jnp.exp(p)).astype(f32)
    cb1 = (-p / r0).astype(f32)
    ca2 = (xi * xi * jnp.exp(2.0 * q)).astype(f32)
    cb2 = (-2.0 * q / r0).astype(f32)
    coef = jnp.concatenate(
        [ca1, cb1, ca2, cb2, offsets.astype(f32), jnp.zeros((1,), f32)])

    mesh = plsc.VectorSubcoreMesh(core_axis_name="c", subcore_axis_name="s")
    sc = functools.partial(
        pl.kernel,
        mesh=mesh,
        compiler_params=pltpu.CompilerParams(
            needs_layout_passes=False, use_tc_tiling_on_sc=True),
        out_type=[
            jax.ShapeDtypeStruct((B * N,), f32),
            jax.ShapeDtypeStruct((B * N,), f32),
        ],
        scratch_types=[
            pltpu.VMEM((CHUNK, M), f32),
            pltpu.VMEM((CHUNK, M), f32),
            pltpu.VMEM((CHUNK, M), jnp.int32),
            pltpu.VMEM((CHUNK, M), jnp.int32),
            pltpu.VMEM((16,), f32),
            pltpu.VMEM((ATOMS_PER_W,), f32),
            pltpu.VMEM((ATOMS_PER_W,), f32),
            pltpu.SemaphoreType.DMA,
            pltpu.SemaphoreType.DMA,
        ],
    )(_sc_stage)
    phi_sum, rho_sum = sc(distances, pair_types, coef)

    phi2 = phi_sum.reshape(B, N)
    rho2 = rho_sum.reshape(B, N)
    nb = jnp.broadcast_to(n_atoms.reshape(B, 1), (B, 128)).astype(jnp.int32)
    fpad = jnp.concatenate(
        [jnp.broadcast_to(F_scale.reshape(2, 1).astype(f32), (2, 128)),
         jnp.zeros((6, 128), f32)])

    out = pl.pallas_call(
        _tc_finale,
        out_shape=jax.ShapeDtypeStruct((B, 128), f32),
    )(phi2, rho2, types, nb, fpad)
    return out[:, :1]


# speedup vs baseline: 1.3976x; 1.3976x over previous
import functools

import jax
import jax.numpy as jnp
from jax import lax
from jax.experimental import pallas as pl
from jax.experimental.pallas import tpu as pltpu
from jax.experimental.pallas import tpu_sc as plsc

B, N, M = 16, 2048, 64
CUTOFF = 6.0
SPLIT = 4
NUM_WORKERS = 32
ATOMS_PER_W = (SPLIT * N) // NUM_WORKERS
CHUNK = 128
NCHUNK = ATOMS_PER_W // CHUNK
GROUPS = CHUNK // 16
UNROLL = 8


def _sc_stage(f_hbm, coef_hbm, phi_hbm, rho_hbm,
              fb_a, fb_b, coef_v, phi_v, rho_v, sem_a, sem_b):
    wid = lax.axis_index("c") * 16 + lax.axis_index("s")
    lane = lax.iota(jnp.int32, 16)
    lane64 = lane * M
    wbase = wid * ATOMS_PER_W * M

    pltpu.sync_copy(coef_hbm, coef_v)

    zero16 = jnp.zeros((16,), jnp.float32)

    def start(buf, sem, c):
        src = f_hbm.at[pl.ds(wbase + c * CHUNK * M, CHUNK * M)]
        pltpu.make_async_copy(src, buf, sem).start()

    def wait(buf, sem):
        pltpu.make_async_copy(f_hbm.at[pl.ds(0, CHUNK * M)], buf, sem).wait()

    def compute(buf, c):
        for g in range(GROUPS):
            gbase = lane64 + g * 16 * M

            def body(jj, acc):
                ap, ar = acc
                for u in range(UNROLL):
                    idx = gbase + (jj * UNROLL + u)
                    fused = plsc.load_gather(buf, [idx])
                    bits = plsc.bitcast(fused, jnp.int32)
                    pt = bits & 3
                    d = plsc.bitcast(bits & (-4), jnp.float32)
                    ca1 = plsc.load_gather(coef_v, [pt])
                    cb1 = plsc.load_gather(coef_v, [pt + 3])
                    ca2 = plsc.load_gather(coef_v, [pt + 6])
                    cb2 = plsc.load_gather(coef_v, [pt + 9])
                    off = plsc.load_gather(coef_v, [pt + 12])
                    m = d < CUTOFF
                    phi = ca1 * jnp.exp(cb1 * d) - off
                    rho = ca2 * jnp.exp(cb2 * d)
                    ap = ap + jnp.where(m, phi, 0.0)
                    ar = ar + jnp.where(m, rho, 0.0)
                return ap, ar

            ap, ar = lax.fori_loop(0, M // UNROLL, body, (zero16, zero16))
            oidx = c * CHUNK + g * 16 + lane
            plsc.store_scatter(phi_v, [oidx], ap)
            plsc.store_scatter(rho_v, [oidx], ar)

    start(fb_a, sem_a, 0)

    def pair_body(c2, carry):
        c = c2 * 2
        start(fb_b, sem_b, c + 1)
        wait(fb_a, sem_a)
        compute(fb_a, c)
        start(fb_a, sem_a, lax.min(c + 2, NCHUNK - 1))
        wait(fb_b, sem_b)
        compute(fb_b, c + 1)
        return carry

    lax.fori_loop(0, NCHUNK // 2, pair_body, 0)
    wait(fb_a, sem_a)

    pltpu.sync_copy(phi_v, phi_hbm.at[pl.ds(wid * ATOMS_PER_W, ATOMS_PER_W)])
    pltpu.sync_copy(rho_v, rho_hbm.at[pl.ds(wid * ATOMS_PER_W, ATOMS_PER_W)])


def _tc_finale(phi_ref, rho_ref, t_ref, n_ref, f_ref, out_ref):
    phi = phi_ref[...]
    rho = rho_ref[...]
    t = t_ref[...]
    f0 = f_ref[0, 0]
    f1 = f_ref[1, 0]
    fs = jnp.where(t == 0, f0, f1)
    emb = -fs * jnp.sqrt(rho + 1e-12)
    col = lax.broadcasted_iota(jnp.int32, (SPLIT, N), 1)
    n = n_ref[:, 0:1]
    amask = col < n
    ae = jnp.where(amask, 0.5 * phi + emb, 0.0)
    e = jnp.sum(ae, axis=1, keepdims=True)
    epa = e / n.astype(jnp.float32)
    out_ref[...] = jnp.broadcast_to(epa, (SPLIT, 128))


def _tc_main(d_ref, pt_ref, t_ref, n_ref, prm_ref, out_ref):
    d3 = d_ref[0].reshape(16, 128, M)
    pt = pt_ref[0].reshape(16, 128, M)
    is0 = pt == 0
    is1 = pt == 1

    def sel(row):
        return jnp.where(
            is0, prm_ref[row, 0],
            jnp.where(is1, prm_ref[row, 1], prm_ref[row, 2]))

    m = (d3 > 0.0) & (d3 < CUTOFF)
    phi = sel(0) * jnp.exp(sel(1) * d3) - sel(4)
    rho = sel(2) * jnp.exp(sel(3) * d3)
    zero = jnp.zeros_like(phi)
    phis = jnp.sum(jnp.where(m, phi, zero), axis=-1)
    rhos = jnp.sum(jnp.where(m, rho, zero), axis=-1)
    t2 = t_ref[0]
    fs = jnp.where(t2 == 0, prm_ref[5, 0], prm_ref[5, 1])
    emb = -fs * jnp.sqrt(rhos + 1e-12)
    ai = (lax.broadcasted_iota(jnp.int32, (16, 128), 0) * 128
          + lax.broadcasted_iota(jnp.int32, (16, 128), 1))
    n = n_ref[0, 0, 0]
    ae = jnp.where(ai < n, 0.5 * phis + emb, 0.0)
    e = jnp.sum(ae)
    out_ref[...] = jnp.full((1, 1, 128), e / n.astype(jnp.float32))


def kernel(types, pair_types, distances, n_atoms, A, p, q, xi, r0, F_scale,
           offsets):
    f32 = jnp.float32
    ca1 = (A * jnp.exp(p)).astype(f32)
    cb1 = (-p / r0).astype(f32)
    ca2 = (xi * xi * jnp.exp(2.0 * q)).astype(f32)
    cb2 = (-2.0 * q / r0).astype(f32)
    coef = jnp.concatenate(
        [ca1, cb1, ca2, cb2, offsets.astype(f32), jnp.zeros((1,), f32)])

    nb = jnp.broadcast_to(n_atoms.reshape(B, 1), (B, 128)).astype(jnp.int32)
    fpad = jnp.concatenate(
        [jnp.broadcast_to(F_scale.reshape(2, 1).astype(f32), (2, 128)),
         jnp.zeros((6, 128), f32)])
    prm = jnp.zeros((8, 128), f32)
    prm = prm.at[0, 0:3].set(ca1).at[1, 0:3].set(cb1)
    prm = prm.at[2, 0:3].set(ca2).at[3, 0:3].set(cb2)
    prm = prm.at[4, 0:3].set(offsets.astype(f32))
    prm = prm.at[5, 0:2].set(F_scale.astype(f32))

    dbits = lax.bitcast_convert_type(distances[:SPLIT].reshape(-1), jnp.int32)
    f1 = lax.bitcast_convert_type(
        (dbits & (-4)) | pair_types[:SPLIT].reshape(-1), f32)

    mesh = plsc.VectorSubcoreMesh(core_axis_name="c", subcore_axis_name="s")
    sc = functools.partial(
        pl.kernel,
        mesh=mesh,
        compiler_params=pltpu.CompilerParams(needs_layout_passes=False),
        out_type=[
            jax.ShapeDtypeStruct((SPLIT * N,), f32),
            jax.ShapeDtypeStruct((SPLIT * N,), f32),
        ],
        scratch_types=[
            pltpu.VMEM((CHUNK * M,), f32),
            pltpu.VMEM((CHUNK * M,), f32),
            pltpu.VMEM((16,), f32),
            pltpu.VMEM((ATOMS_PER_W,), f32),
            pltpu.VMEM((ATOMS_PER_W,), f32),
            pltpu.SemaphoreType.DMA,
            pltpu.SemaphoreType.DMA,
        ],
    )(_sc_stage)
    phi_sum, rho_sum = sc(f1, coef)

    e_sc = pl.pallas_call(
        _tc_finale,
        out_shape=jax.ShapeDtypeStruct((SPLIT, 128), f32),
    )(phi_sum.reshape(SPLIT, N), rho_sum.reshape(SPLIT, N),
      types[:SPLIT], nb[:SPLIT], fpad)

    t3 = types.reshape(B, 16, 128)
    n3 = nb.reshape(B, 1, 128)
    e_tc = pl.pallas_call(
        _tc_main,
        grid=(B - SPLIT,),
        in_specs=[
            pl.BlockSpec((1, N, M), lambda i: (i + SPLIT, 0, 0)),
            pl.BlockSpec((1, N, M), lambda i: (i + SPLIT, 0, 0)),
            pl.BlockSpec((1, 16, 128), lambda i: (i + SPLIT, 0, 0)),
            pl.BlockSpec((1, 1, 128), lambda i: (i + SPLIT, 0, 0)),
            pl.BlockSpec((8, 128), lambda i: (0, 0)),
        ],
        out_specs=pl.BlockSpec((1, 1, 128), lambda i: (i, 0, 0)),
        out_shape=jax.ShapeDtypeStruct((B - SPLIT, 1, 128), f32),
    )(distances, pair_types, t3, n3, prm)

    return jnp.concatenate([e_sc[:, :1], e_tc[:, 0, :1]], axis=0)


# speedup vs baseline: 2.9524x; 2.1125x over previous
import functools

import jax
import jax.numpy as jnp
from jax import lax
from jax.experimental import pallas as pl
from jax.experimental.pallas import tpu as pltpu
from jax.experimental.pallas import tpu_sc as plsc

B, N, M = 16, 2048, 64
CUTOFF = 6.0
SPLIT = 6
NUM_WORKERS = 32
NBLK = 128
BLOCKS_PER_W = (SPLIT * N) // (NUM_WORKERS * NBLK)
ATOMS_PER_W = BLOCKS_PER_W * NBLK
UNROLL = 8
SROWS = SPLIT * N // 128


def _sc_stage(f_hbm, coef_hbm, phi_hbm, rho_hbm,
              fb_a, fb_b, coef_v, phi_v, rho_v, sem_a, sem_b):
    wid = lax.axis_index("c") * 16 + lax.axis_index("s")

    pltpu.sync_copy(coef_hbm, coef_v)

    zero16 = jnp.zeros((16,), jnp.float32)
    bufs = [(fb_a, sem_a), (fb_b, sem_b)]

    def start(bs, c):
        buf, sem = bs
        blk = wid * BLOCKS_PER_W + c
        src = f_hbm.at[blk // 16, :, pl.ds((blk % 16) * NBLK, NBLK)]
        pltpu.make_async_copy(src, buf, sem).start()

    def wait(bs):
        buf, sem = bs
        pltpu.make_async_copy(f_hbm.at[0, :, pl.ds(0, NBLK)], buf, sem).wait()

    def compute(bs, c):
        buf, _ = bs
        for v in range(NBLK // 16):

            def body(jj, acc):
                ap, ar = acc
                for u in range(UNROLL):
                    fused = buf[jj * UNROLL + u, pl.ds(v * 16, 16)]
                    bits = plsc.bitcast(fused, jnp.int32)
                    pt = bits & 3
                    d = plsc.bitcast(bits & (-4), jnp.float32)
                    ca1 = plsc.load_gather(coef_v, [pt])
                    cb1 = plsc.load_gather(coef_v, [pt + 3])
                    ca2 = plsc.load_gather(coef_v, [pt + 6])
                    cb2 = plsc.load_gather(coef_v, [pt + 9])
                    off = plsc.load_gather(coef_v, [pt + 12])
                    m = d < CUTOFF
                    phi = ca1 * jnp.exp(cb1 * d) - off
                    rho = ca2 * jnp.exp(cb2 * d)
                    ap = ap + jnp.where(m, phi, 0.0)
                    ar = ar + jnp.where(m, rho, 0.0)
                return ap, ar

            ap, ar = lax.fori_loop(0, M // UNROLL, body, (zero16, zero16))
            off0 = c * NBLK + v * 16
            phi_v[pl.ds(off0, 16)] = ap
            rho_v[pl.ds(off0, 16)] = ar

    start(bufs[0], 0)
    if BLOCKS_PER_W > 1:
        start(bufs[1], 1)
    for c in range(BLOCKS_PER_W):
        wait(bufs[c % 2])
        compute(bufs[c % 2], c)
        if c + 2 < BLOCKS_PER_W:
            start(bufs[c % 2], c + 2)

    pltpu.sync_copy(phi_v, phi_hbm.at[pl.ds(wid * ATOMS_PER_W, ATOMS_PER_W)])
    pltpu.sync_copy(rho_v, rho_hbm.at[pl.ds(wid * ATOMS_PER_W, ATOMS_PER_W)])


def _tc_finale(phi_ref, rho_ref, t_ref, n_ref, f_ref, out_ref):
    phi = phi_ref[...]
    rho = rho_ref[...]
    t = t_ref[...]
    f0 = f_ref[0, 0]
    f1 = f_ref[1, 0]
    fs = jnp.where(t == 0, f0, f1)
    emb = -fs * jnp.sqrt(rho + 1e-12)
    nbig = n_ref[...]
    ai = ((lax.broadcasted_iota(jnp.int32, (SROWS, 128), 0) & 15) * 128
          + lax.broadcasted_iota(jnp.int32, (SROWS, 128), 1))
    ae = jnp.where(ai < nbig, 0.5 * phi + emb, 0.0)
    row_sum = jnp.sum(ae, axis=1)
    e = jnp.sum(row_sum.reshape(SPLIT, 16), axis=1, keepdims=True)
    nf = nbig.reshape(SPLIT, 16, 128)[:, 0, 0:1].astype(jnp.float32)
    out_ref[...] = jnp.broadcast_to(e / nf, (SPLIT, 128))


def _tc_main(d_ref, pt_ref, t_ref, n_ref, prm_ref, out_ref):
    d = d_ref[0]
    pt = pt_ref[0]
    is0 = pt == 0
    is1 = pt == 1

    def sel(row):
        return jnp.where(
            is0, prm_ref[row, 0],
            jnp.where(is1, prm_ref[row, 1], prm_ref[row, 2]))

    m = (d > 0.0) & (d < CUTOFF)
    phi = sel(0) * jnp.exp(sel(1) * d) - sel(4)
    rho = sel(2) * jnp.exp(sel(3) * d)
    zero = jnp.zeros_like(phi)
    phis = jnp.sum(jnp.where(m, phi, zero), axis=0, keepdims=True)
    rhos = jnp.sum(jnp.where(m, rho, zero), axis=0, keepdims=True)
    t2 = t_ref[0]
    fs = jnp.where(t2 == 0, prm_ref[5, 0], prm_ref[5, 1])
    emb = -fs * jnp.sqrt(rhos + 1e-12)
    ai = lax.broadcasted_iota(jnp.int32, (1, N), 1)
    n = n_ref[0, 0, 0]
    ae = jnp.where(ai < n, 0.5 * phis + emb, 0.0)
    e = jnp.sum(ae)
    out_ref[...] = jnp.full((1, 1, 128), e / n.astype(jnp.float32))


def kernel(types, pair_types, distances, n_atoms, A, p, q, xi, r0, F_scale,
           offsets):
    f32 = jnp.float32
    ca1 = (A * jnp.exp(p)).astype(f32)
    cb1 = (-p / r0).astype(f32)
    ca2 = (xi * xi * jnp.exp(2.0 * q)).astype(f32)
    cb2 = (-2.0 * q / r0).astype(f32)
    coef = jnp.concatenate(
        [ca1, cb1, ca2, cb2, offsets.astype(f32), jnp.zeros((1,), f32)])

    nb = jnp.broadcast_to(n_atoms.reshape(B, 1), (B, 128)).astype(jnp.int32)
    fpad = jnp.concatenate(
        [jnp.broadcast_to(F_scale.reshape(2, 1).astype(f32), (2, 128)),
         jnp.zeros((6, 128), f32)])
    prm = jnp.zeros((8, 128), f32)
    prm = prm.at[0, 0:3].set(ca1).at[1, 0:3].set(cb1)
    prm = prm.at[2, 0:3].set(ca2).at[3, 0:3].set(cb2)
    prm = prm.at[4, 0:3].set(offsets.astype(f32))
    prm = prm.at[5, 0:2].set(F_scale.astype(f32))

    dT = jnp.transpose(distances, (0, 2, 1))
    ptT = jnp.transpose(pair_types, (0, 2, 1))

    dbits = lax.bitcast_convert_type(dT[:SPLIT], jnp.int32)
    fusedT = lax.bitcast_convert_type((dbits & (-4)) | ptT[:SPLIT], f32)

    mesh = plsc.VectorSubcoreMesh(core_axis_name="c", subcore_axis_name="s")
    sc = functools.partial(
        pl.kernel,
        mesh=mesh,
        compiler_params=pltpu.CompilerParams(needs_layout_passes=False),
        out_type=[
            jax.ShapeDtypeStruct((SPLIT * N,), f32),
            jax.ShapeDtypeStruct((SPLIT * N,), f32),
        ],
        scratch_types=[
            pltpu.VMEM((M, NBLK), f32),
            pltpu.VMEM((M, NBLK), f32),
            pltpu.VMEM((16,), f32),
            pltpu.VMEM((ATOMS_PER_W,), f32),
            pltpu.VMEM((ATOMS_PER_W,), f32),
            pltpu.SemaphoreType.DMA,
            pltpu.SemaphoreType.DMA,
        ],
    )(_sc_stage)
    phi_sum, rho_sum = sc(fusedT, coef)

    t64 = types[:SPLIT].reshape(SROWS, 128)
    n64 = jnp.broadcast_to(
        n_atoms[:SPLIT].astype(jnp.int32).reshape(SPLIT, 1, 1),
        (SPLIT, 16, 128)).reshape(SROWS, 128)
    e_sc = pl.pallas_call(
        _tc_finale,
        out_shape=jax.ShapeDtypeStruct((SPLIT, 128), f32),
    )(phi_sum.reshape(SROWS, 128), rho_sum.reshape(SROWS, 128),
      t64, n64, fpad)

    t3 = types.reshape(B, 1, N)
    n3 = nb.reshape(B, 1, 128)
    e_tc = pl.pallas_call(
        _tc_main,
        grid=(B - SPLIT,),
        in_specs=[
            pl.BlockSpec((1, M, N), lambda i: (i + SPLIT, 0, 0)),
            pl.BlockSpec((1, M, N), lambda i: (i + SPLIT, 0, 0)),
            pl.BlockSpec((1, 1, N), lambda i: (i + SPLIT, 0, 0)),
            pl.BlockSpec((1, 1, 128), lambda i: (i + SPLIT, 0, 0)),
            pl.BlockSpec((8, 128), lambda i: (0, 0)),
        ],
        out_specs=pl.BlockSpec((1, 1, 128), lambda i: (i, 0, 0)),
        out_shape=jax.ShapeDtypeStruct((B - SPLIT, 1, 128), f32),
    )(dT, ptT, t3, n3, prm)

    return jnp.concatenate([e_sc[:, :1], e_tc[:, 0, :1]], axis=0)
